# baseline (speedup 1.0000x reference)
import functools

import jax
import jax.numpy as jnp
from jax import lax
from jax.experimental import pallas as pl
from jax.experimental.pallas import tpu as pltpu
from jax.experimental.pallas import tpu_sc as plsc

_NC = 2
_NS = 16
_NW = _NC * _NS

_B = 4096 * 200
_D = 64
_CH = 128
_K = 4
_ROWS_G = _K * _CH
_PER_W = _B // _NW
_GROUPS = _PER_W // _ROWS_G


def _body(xr, emb, out, idx0, idx1, rows0, rows1,
          isem0, isem1, gsem0, gsem1, ssem0, ssem1):
  wid = lax.axis_index("s") * _NC + lax.axis_index("c")
  base = wid * _GROUPS

  def load_idx(g, dst, sem):
    return pltpu.async_copy(xr.at[base + g], dst, sem)

  def gathers(idx_v, dst, sem):
    for k in range(_K):
      pltpu.async_copy(emb.at[idx_v.at[k]], dst.at[pl.ds(k * _CH, _CH)], sem)

  def drain_g(idx_v, dst, sem):
    for k in range(_K):
      pltpu.make_async_copy(emb.at[idx_v.at[k]],
                            dst.at[pl.ds(k * _CH, _CH)], sem).wait()

  def store(g, src, sem):
    return pltpu.async_copy(src, out.at[base + g], sem)

  def drain_s(g, src, sem):
    pltpu.make_async_copy(src, out.at[base + g], sem).wait()

  load_idx(0, idx0, isem0)
  load_idx(1, idx1, isem1)
  pltpu.make_async_copy(xr.at[base], idx0, isem0).wait()
  gathers(idx0, rows0, gsem0)

  def step(p, carry):
    a = 2 * p
    b = a + 1

    pltpu.make_async_copy(xr.at[base + b], idx1, isem1).wait()

    @pl.when(p > 0)
    def _():
      drain_s(b - 2, rows1, ssem1)

    gathers(idx1, rows1, gsem1)

    drain_g(idx0, rows0, gsem0)

    @pl.when(p < _GROUPS // 2 - 1)
    def _():
      load_idx(a + 2, idx0, isem0)

    store(a, rows0, ssem0)
    drain_s(a, rows0, ssem0)

    @pl.when(p < _GROUPS // 2 - 1)
    def _():
      pltpu.make_async_copy(xr.at[base + a + 2], idx0, isem0).wait()
      gathers(idx0, rows0, gsem0)

    drain_g(idx1, rows1, gsem1)

    @pl.when(p < _GROUPS // 2 - 1)
    def _():
      load_idx(b + 2, idx1, isem1)

    store(b, rows1, ssem1)
    return carry

  lax.fori_loop(0, _GROUPS // 2, step, 0)
  drain_s(_GROUPS - 1, rows1, ssem1)


@jax.jit
def kernel(x, embedding):
  xr = x.reshape(_NW * _GROUPS, _K, _CH).astype(jnp.int32)
  mesh = plsc.VectorSubcoreMesh(
      core_axis_name="c", subcore_axis_name="s",
      num_cores=_NC, num_subcores=_NS)
  out = pl.kernel(
      _body,
      out_type=jax.ShapeDtypeStruct((_NW * _GROUPS, _ROWS_G, _D),
                                    jnp.float32),
      mesh=mesh,
      compiler_params=pltpu.CompilerParams(use_tc_tiling_on_sc=False),
      scratch_types=[
          pltpu.VMEM((_K, _CH), jnp.int32),
          pltpu.VMEM((_K, _CH), jnp.int32),
          pltpu.VMEM((_ROWS_G, _D), jnp.float32),
          pltpu.VMEM((_ROWS_G, _D), jnp.float32),
          pltpu.SemaphoreType.DMA,
          pltpu.SemaphoreType.DMA,
          pltpu.SemaphoreType.DMA,
          pltpu.SemaphoreType.DMA,
          pltpu.SemaphoreType.DMA,
          pltpu.SemaphoreType.DMA,
      ],
  )(xr, embedding)
  return out.reshape(4096, 200, _D)


# speedup vs baseline: 1.0008x; 1.0008x over previous
import functools

import jax
import jax.numpy as jnp
from jax import lax
from jax.experimental import pallas as pl
from jax.experimental.pallas import tpu as pltpu
from jax.experimental.pallas import tpu_sc as plsc

_NC = 2
_NS = 16
_NW = _NC * _NS

_B = 4096 * 200
_D = 64
_CH = 128
_K = 4
_ROWS_G = _K * _CH
_PER_W = _B // _NW
_GROUPS = _PER_W // _ROWS_G


def _body(xr, emb, out, idx0, idx1, rows0, rows1,
          isem0, isem1, gsem0, gsem1, ssem0, ssem1):
  wid = lax.axis_index("s") * _NC + lax.axis_index("c")
  base = wid * _PER_W

  def load_idx(g, dst, sem):
    return pltpu.async_copy(xr.at[pl.ds(base + g * _ROWS_G, _ROWS_G)],
                            dst, sem)

  def wait_idx(g, dst, sem):
    pltpu.make_async_copy(xr.at[pl.ds(base + g * _ROWS_G, _ROWS_G)],
                          dst, sem).wait()

  def gathers(idx_v, dst, sem):
    for k in range(_K):
      pltpu.async_copy(emb.at[idx_v.at[pl.ds(k * _CH, _CH)]],
                       dst.at[pl.ds(k * _CH, _CH)], sem)

  def drain_g(idx_v, dst, sem):
    for k in range(_K):
      pltpu.make_async_copy(emb.at[idx_v.at[pl.ds(k * _CH, _CH)]],
                            dst.at[pl.ds(k * _CH, _CH)], sem).wait()

  def store(g, src, sem):
    return pltpu.async_copy(src, out.at[pl.ds(base + g * _ROWS_G, _ROWS_G)],
                            sem)

  def drain_s(g, src, sem):
    pltpu.make_async_copy(src, out.at[pl.ds(base + g * _ROWS_G, _ROWS_G)],
                          sem).wait()

  load_idx(0, idx0, isem0)
  load_idx(1, idx1, isem1)
  wait_idx(0, idx0, isem0)
  gathers(idx0, rows0, gsem0)

  def step(p, carry):
    a = 2 * p
    b = a + 1

    wait_idx(b, idx1, isem1)

    @pl.when(p > 0)
    def _():
      drain_s(b - 2, rows1, ssem1)

    gathers(idx1, rows1, gsem1)

    drain_g(idx0, rows0, gsem0)

    @pl.when(p < _GROUPS // 2 - 1)
    def _():
      load_idx(a + 2, idx0, isem0)

    store(a, rows0, ssem0)
    drain_s(a, rows0, ssem0)

    @pl.when(p < _GROUPS // 2 - 1)
    def _():
      wait_idx(a + 2, idx0, isem0)
      gathers(idx0, rows0, gsem0)

    drain_g(idx1, rows1, gsem1)

    @pl.when(p < _GROUPS // 2 - 1)
    def _():
      load_idx(b + 2, idx1, isem1)

    store(b, rows1, ssem1)
    return carry

  lax.fori_loop(0, _GROUPS // 2, step, 0)
  drain_s(_GROUPS - 1, rows1, ssem1)


@jax.jit
def kernel(x, embedding):
  xf = x.reshape(_B).astype(jnp.int32)
  mesh = plsc.VectorSubcoreMesh(
      core_axis_name="c", subcore_axis_name="s",
      num_cores=_NC, num_subcores=_NS)
  out = pl.kernel(
      _body,
      out_type=jax.ShapeDtypeStruct((_B, _D), jnp.float32),
      mesh=mesh,
      compiler_params=pltpu.CompilerParams(use_tc_tiling_on_sc=False),
      scratch_types=[
          pltpu.VMEM((_ROWS_G,), jnp.int32),
          pltpu.VMEM((_ROWS_G,), jnp.int32),
          pltpu.VMEM((_ROWS_G, _D), jnp.float32),
          pltpu.VMEM((_ROWS_G, _D), jnp.float32),
          pltpu.SemaphoreType.DMA,
          pltpu.SemaphoreType.DMA,
          pltpu.SemaphoreType.DMA,
          pltpu.SemaphoreType.DMA,
          pltpu.SemaphoreType.DMA,
          pltpu.SemaphoreType.DMA,
      ],
  )(xf, embedding)
  return out.reshape(4096, 200, _D)


# speedup vs baseline: 1.2227x; 1.2217x over previous
import functools

import jax
import jax.numpy as jnp
from jax import lax
from jax.experimental import pallas as pl
from jax.experimental.pallas import tpu as pltpu
from jax.experimental.pallas import tpu_sc as plsc

_NC = 2
_NS = 16
_NW = _NC * _NS

_B = 4096 * 200
_D = 64
_DP = 128
_CH = 128
_K = 2
_ROWS_G = _K * _CH
_PER_W = _B // _NW
_GROUPS = _PER_W // _ROWS_G


def _body(xr, emb, out, idx0, idx1, rows0, rows1,
          isem0, isem1, gsem0, gsem1, ssem0, ssem1):
  wid = lax.axis_index("s") * _NC + lax.axis_index("c")
  base = wid * _PER_W

  def load_idx(g, dst, sem):
    return pltpu.async_copy(xr.at[pl.ds(base + g * _ROWS_G, _ROWS_G)],
                            dst, sem)

  def wait_idx(g, dst, sem):
    pltpu.make_async_copy(xr.at[pl.ds(base + g * _ROWS_G, _ROWS_G)],
                          dst, sem).wait()

  def gathers(idx_v, dst, sem):
    for k in range(_K):
      pltpu.async_copy(emb.at[idx_v.at[pl.ds(k * _CH, _CH)]],
                       dst.at[pl.ds(k * _CH, _CH)], sem)

  def drain_g(idx_v, dst, sem):
    for k in range(_K):
      pltpu.make_async_copy(emb.at[idx_v.at[pl.ds(k * _CH, _CH)]],
                            dst.at[pl.ds(k * _CH, _CH)], sem).wait()

  def store(g, src, sem):
    return pltpu.async_copy(src, out.at[pl.ds(base + g * _ROWS_G, _ROWS_G)],
                            sem)

  def drain_s(g, src, sem):
    pltpu.make_async_copy(src, out.at[pl.ds(base + g * _ROWS_G, _ROWS_G)],
                          sem).wait()

  load_idx(0, idx0, isem0)
  load_idx(1, idx1, isem1)
  wait_idx(0, idx0, isem0)
  gathers(idx0, rows0, gsem0)

  def step(p, carry):
    a = 2 * p
    b = a + 1

    wait_idx(b, idx1, isem1)

    @pl.when(p > 0)
    def _():
      drain_s(b - 2, rows1, ssem1)

    gathers(idx1, rows1, gsem1)

    drain_g(idx0, rows0, gsem0)

    @pl.when(p < _GROUPS // 2 - 1)
    def _():
      load_idx(a + 2, idx0, isem0)

    store(a, rows0, ssem0)
    drain_s(a, rows0, ssem0)

    @pl.when(p < _GROUPS // 2 - 1)
    def _():
      wait_idx(a + 2, idx0, isem0)
      gathers(idx0, rows0, gsem0)

    drain_g(idx1, rows1, gsem1)

    @pl.when(p < _GROUPS // 2 - 1)
    def _():
      load_idx(b + 2, idx1, isem1)

    store(b, rows1, ssem1)
    return carry

  lax.fori_loop(0, _GROUPS // 2, step, 0)
  drain_s(_GROUPS - 1, rows1, ssem1)


@jax.jit
def kernel(x, embedding):
  xf = x.reshape(_B).astype(jnp.int32)
  embp = jnp.pad(embedding, ((0, 0), (0, _DP - _D)))
  mesh = plsc.VectorSubcoreMesh(
      core_axis_name="c", subcore_axis_name="s",
      num_cores=_NC, num_subcores=_NS)
  out = pl.kernel(
      _body,
      out_type=jax.ShapeDtypeStruct((_B, _DP), jnp.float32),
      mesh=mesh,
      compiler_params=pltpu.CompilerParams(use_tc_tiling_on_sc=False),
      scratch_types=[
          pltpu.VMEM((_ROWS_G,), jnp.int32),
          pltpu.VMEM((_ROWS_G,), jnp.int32),
          pltpu.VMEM((_ROWS_G, _DP), jnp.float32),
          pltpu.VMEM((_ROWS_G, _DP), jnp.float32),
          pltpu.SemaphoreType.DMA,
          pltpu.SemaphoreType.DMA,
          pltpu.SemaphoreType.DMA,
          pltpu.SemaphoreType.DMA,
          pltpu.SemaphoreType.DMA,
          pltpu.SemaphoreType.DMA,
      ],
  )(xf, embp)
  return out[:, :_D].reshape(4096, 200, _D)


# speedup vs baseline: 1.2258x; 1.0025x over previous
import functools

import jax
import jax.numpy as jnp
from jax import lax
from jax.experimental import pallas as pl
from jax.experimental.layout import Format, Layout
from jax.experimental.pallas import tpu as pltpu
from jax.experimental.pallas import tpu_sc as plsc

_NC = 2
_NS = 16
_NW = _NC * _NS

_B = 4096 * 200
_D = 64
_DP = 128
_CH = 128
_K = 2
_ROWS_G = _K * _CH
_PER_W = _B // _NW
_GROUPS = _PER_W // _ROWS_G


def _body(xr, emb, out, idx0, idx1, rows0, rows1,
          isem0, isem1, gsem0, gsem1, ssem0, ssem1):
  wid = lax.axis_index("s") * _NC + lax.axis_index("c")
  base = wid * _PER_W

  def load_idx(g, dst, sem):
    return pltpu.async_copy(xr.at[pl.ds(base + g * _ROWS_G, _ROWS_G)],
                            dst, sem)

  def wait_idx(g, dst, sem):
    pltpu.make_async_copy(xr.at[pl.ds(base + g * _ROWS_G, _ROWS_G)],
                          dst, sem).wait()

  def gathers(idx_v, dst, sem):
    for k in range(_K):
      pltpu.async_copy(emb.at[idx_v.at[pl.ds(k * _CH, _CH)]],
                       dst.at[pl.ds(k * _CH, _CH)], sem)

  def drain_g(idx_v, dst, sem):
    for k in range(_K):
      pltpu.make_async_copy(emb.at[idx_v.at[pl.ds(k * _CH, _CH)]],
                            dst.at[pl.ds(k * _CH, _CH)], sem).wait()

  def store(g, src, sem):
    return pltpu.async_copy(src, out.at[pl.ds(base + g * _ROWS_G, _ROWS_G)],
                            sem)

  def drain_s(g, src, sem):
    pltpu.make_async_copy(src, out.at[pl.ds(base + g * _ROWS_G, _ROWS_G)],
                          sem).wait()

  load_idx(0, idx0, isem0)
  load_idx(1, idx1, isem1)
  wait_idx(0, idx0, isem0)
  gathers(idx0, rows0, gsem0)

  def step(p, carry):
    a = 2 * p
    b = a + 1

    wait_idx(b, idx1, isem1)

    @pl.when(p > 0)
    def _():
      drain_s(b - 2, rows1, ssem1)

    gathers(idx1, rows1, gsem1)

    drain_g(idx0, rows0, gsem0)

    @pl.when(p < _GROUPS // 2 - 1)
    def _():
      load_idx(a + 2, idx0, isem0)

    store(a, rows0, ssem0)
    drain_s(a, rows0, ssem0)

    @pl.when(p < _GROUPS // 2 - 1)
    def _():
      wait_idx(a + 2, idx0, isem0)
      gathers(idx0, rows0, gsem0)

    drain_g(idx1, rows1, gsem1)

    @pl.when(p < _GROUPS // 2 - 1)
    def _():
      load_idx(b + 2, idx1, isem1)

    store(b, rows1, ssem1)
    return carry

  lax.fori_loop(0, _GROUPS // 2, step, 0)
  drain_s(_GROUPS - 1, rows1, ssem1)


def _impl(x, embedding):
  xf = x.reshape(_B).astype(jnp.int32)
  embp = jnp.pad(embedding, ((0, 0), (0, _DP - _D)))
  mesh = plsc.VectorSubcoreMesh(
      core_axis_name="c", subcore_axis_name="s",
      num_cores=_NC, num_subcores=_NS)
  out = pl.kernel(
      _body,
      out_type=jax.ShapeDtypeStruct((_B, _DP), jnp.float32),
      mesh=mesh,
      compiler_params=pltpu.CompilerParams(use_tc_tiling_on_sc=False),
      scratch_types=[
          pltpu.VMEM((_ROWS_G,), jnp.int32),
          pltpu.VMEM((_ROWS_G,), jnp.int32),
          pltpu.VMEM((_ROWS_G, _DP), jnp.float32),
          pltpu.VMEM((_ROWS_G, _DP), jnp.float32),
          pltpu.SemaphoreType.DMA,
          pltpu.SemaphoreType.DMA,
          pltpu.SemaphoreType.DMA,
          pltpu.SemaphoreType.DMA,
          pltpu.SemaphoreType.DMA,
          pltpu.SemaphoreType.DMA,
      ],
  )(xf, embp)
  return out[:, :_D].reshape(4096, 200, _D)


_jfn = None


def kernel(x, embedding):
  global _jfn
  if _jfn is None:
    try:
      tpus = jax.devices("tpu")
    except Exception:
      tpus = [d for d in jax.devices() if d.platform == "tpu"]
    if tpus:
      sh = jax.sharding.SingleDeviceSharding(tpus[0])
      _jfn = jax.jit(
          _impl,
          out_shardings=Format(Layout(major_to_minor=(0, 1, 2)), sh))
    else:
      _jfn = jax.jit(_impl)
  return _jfn(x, embedding)


# speedup vs baseline: 1.3346x; 1.0887x over previous
import functools

import jax
import jax.numpy as jnp
from jax import lax
from jax.experimental import pallas as pl
from jax.experimental.pallas import tpu as pltpu
from jax.experimental.pallas import tpu_sc as plsc

_NC = 2
_NS = 16
_NW = _NC * _NS

_B = 4096 * 200
_D = 64
_DP = 128
_CH = 128
_K = 4
_ROWS_G = _K * _CH
_PER_W = _B // _NW
_GROUPS = _PER_W // _ROWS_G


def _body(xr, emb, out, idx0, idx1, rows0, rows1,
          isem0, isem1, gsem0, gsem1, ssem0, ssem1):
  wid = lax.axis_index("s") * _NC + lax.axis_index("c")
  base = wid * _PER_W

  def load_idx(g, dst, sem):
    return pltpu.async_copy(xr.at[pl.ds(base + g * _ROWS_G, _ROWS_G)],
                            dst, sem)

  def wait_idx(g, dst, sem):
    pltpu.make_async_copy(xr.at[pl.ds(base + g * _ROWS_G, _ROWS_G)],
                          dst, sem).wait()

  def gathers(idx_v, dst, sem):
    for k in range(_K):
      pltpu.async_copy(emb.at[idx_v.at[pl.ds(k * _CH, _CH)]],
                       dst.at[pl.ds(k * _CH, _CH)], sem)

  def drain_g(idx_v, dst, sem):
    for k in range(_K):
      pltpu.make_async_copy(emb.at[idx_v.at[pl.ds(k * _CH, _CH)]],
                            dst.at[pl.ds(k * _CH, _CH)], sem).wait()

  def store(g, src, sem):
    return pltpu.async_copy(
        src, out.at[pl.ds(base + g * _ROWS_G, _ROWS_G), pl.ds(0, _D)], sem)

  def drain_s(g, src, sem):
    pltpu.make_async_copy(
        src, out.at[pl.ds(base + g * _ROWS_G, _ROWS_G), pl.ds(0, _D)],
        sem).wait()

  load_idx(0, idx0, isem0)
  load_idx(1, idx1, isem1)
  wait_idx(0, idx0, isem0)
  gathers(idx0, rows0, gsem0)

  def step(p, carry):
    a = 2 * p
    b = a + 1

    wait_idx(b, idx1, isem1)

    @pl.when(p > 0)
    def _():
      drain_s(b - 2, rows1, ssem1)

    gathers(idx1, rows1, gsem1)

    drain_g(idx0, rows0, gsem0)

    @pl.when(p < _GROUPS // 2 - 1)
    def _():
      load_idx(a + 2, idx0, isem0)

    store(a, rows0, ssem0)
    drain_s(a, rows0, ssem0)

    @pl.when(p < _GROUPS // 2 - 1)
    def _():
      wait_idx(a + 2, idx0, isem0)
      gathers(idx0, rows0, gsem0)

    drain_g(idx1, rows1, gsem1)

    @pl.when(p < _GROUPS // 2 - 1)
    def _():
      load_idx(b + 2, idx1, isem1)

    store(b, rows1, ssem1)
    return carry

  lax.fori_loop(0, _GROUPS // 2, step, 0)
  drain_s(_GROUPS - 1, rows1, ssem1)


def _impl(x, embedding):
  xf = x.reshape(_B).astype(jnp.int32)
  mesh = plsc.VectorSubcoreMesh(
      core_axis_name="c", subcore_axis_name="s",
      num_cores=_NC, num_subcores=_NS)
  out = pl.kernel(
      _body,
      out_type=jax.ShapeDtypeStruct((_B, _DP), jnp.float32),
      mesh=mesh,
      compiler_params=pltpu.CompilerParams(use_tc_tiling_on_sc=False),
      scratch_types=[
          pltpu.VMEM((_ROWS_G,), jnp.int32),
          pltpu.VMEM((_ROWS_G,), jnp.int32),
          pltpu.VMEM((_ROWS_G, _D), jnp.float32),
          pltpu.VMEM((_ROWS_G, _D), jnp.float32),
          pltpu.SemaphoreType.DMA,
          pltpu.SemaphoreType.DMA,
          pltpu.SemaphoreType.DMA,
          pltpu.SemaphoreType.DMA,
          pltpu.SemaphoreType.DMA,
          pltpu.SemaphoreType.DMA,
      ],
  )(xf, embedding)
  return out[:, :_D].reshape(4096, 200, _D)


kernel = jax.jit(_impl)
